# Initial kernel scaffold; baseline (speedup 1.0000x reference)
import jax
import jax.numpy as jnp
from jax.experimental import pallas as pl


def kernel(pts3d_xyz, target_pts3d, target_valid):
    raise NotImplementedError("write your pallas kernel here")


# speedup vs baseline: 1.4906x; 1.4906x over previous
import jax
import jax.numpy as jnp
from jax.experimental import pallas as pl
from jax.experimental.pallas import tpu as pltpu

B = 4
N = 4096
TILE_R = 512
NR = N // TILE_R
BIG = 1e10


def _chamfer_body(pr_ref, gt_ref, val_ref, out_ref, colmin_ref, fwd_ref):
    b = pl.program_id(0)
    r = pl.program_id(1)
    nr = pl.num_programs(1)

    @pl.when(jnp.logical_and(b == 0, r == 0))
    def _init_out():
        out_ref[0, 0] = jnp.float32(0.0)

    @pl.when(r == 0)
    def _init_batch():
        colmin_ref[...] = jnp.full(colmin_ref.shape, BIG, jnp.float32)
        fwd_ref[0, 0] = jnp.float32(0.0)

    pr = pr_ref[0]
    gt = gt_ref[0]
    val = val_ref[0]

    d = None
    for k in range(3):
        diff = pr[:, k:k + 1] - gt[k:k + 1, :]
        sq = diff * diff
        d = sq if d is None else d + sq

    dm = jnp.where(val > 0.0, d, BIG)
    fwd_ref[0, 0] += jnp.sum(jnp.min(dm, axis=1, keepdims=True))
    colmin_ref[...] = jnp.minimum(colmin_ref[...],
                                  jnp.min(d, axis=0, keepdims=True))

    @pl.when(r == nr - 1)
    def _finish_batch():
        lf = fwd_ref[0, 0] / jnp.float32(N)
        vsum = jnp.sum(val)
        lb = jnp.sum(colmin_ref[...] * val) / jnp.maximum(vsum, 1.0)
        out_ref[0, 0] += (jnp.float32(2.0) / B) * (lf + lb)


def kernel(pts3d_xyz, target_pts3d, target_valid):
    gt_t = jnp.swapaxes(target_pts3d, 1, 2)
    val_f = target_valid.astype(jnp.float32)[:, None, :]

    out = pl.pallas_call(
        _chamfer_body,
        grid=(B, NR),
        in_specs=[
            pl.BlockSpec((1, TILE_R, 3), lambda b, r: (b, r, 0)),
            pl.BlockSpec((1, 3, N), lambda b, r: (b, 0, 0)),
            pl.BlockSpec((1, 1, N), lambda b, r: (b, 0, 0)),
        ],
        out_specs=pl.BlockSpec(memory_space=pltpu.SMEM),
        out_shape=jax.ShapeDtypeStruct((1, 1), jnp.float32),
        scratch_shapes=[
            pltpu.VMEM((1, N), jnp.float32),
            pltpu.SMEM((1, 1), jnp.float32),
        ],
        compiler_params=pltpu.CompilerParams(
            dimension_semantics=("arbitrary", "arbitrary"),
        ),
    )(pts3d_xyz, gt_t, val_f)
    return out[0, 0]


# speedup vs baseline: 2.9823x; 2.0007x over previous
import jax
import jax.numpy as jnp
from jax.experimental import pallas as pl
from jax.experimental.pallas import tpu as pltpu

B = 4
N = 4096
TILE_R = 512
NR = N // TILE_R
BIG = 1e10


def _chamfer_body(pr_ref, gt_ref, val_ref, out_ref, colmin_ref, fwd_ref):
    b = pl.program_id(0)
    r = pl.program_id(1)
    nr = pl.num_programs(1)

    @pl.when(jnp.logical_and(b == 0, r == 0))
    def _init_out():
        out_ref[0, 0] = jnp.float32(0.0)

    @pl.when(r == 0)
    def _init_batch():
        colmin_ref[...] = jnp.full(colmin_ref.shape, BIG, jnp.float32)
        fwd_ref[0, 0] = jnp.float32(0.0)

    pr = pr_ref[0]
    gt = gt_ref[0]
    val = val_ref[0]

    pn = jnp.sum(pr * pr, axis=1, keepdims=True)
    gn = jnp.sum(gt * gt, axis=0, keepdims=True)
    c2 = jax.lax.dot_general(pr * jnp.float32(-2.0), gt,
                             (((1,), (0,)), ((), ())),
                             preferred_element_type=jnp.float32)
    d = (c2 + pn) + gn

    pen = (jnp.float32(1.0) - val) * jnp.float32(BIG)
    fwd_ref[0, 0] += jnp.sum(jnp.min(d + pen, axis=1, keepdims=True))
    colmin_ref[...] = jnp.minimum(colmin_ref[...],
                                  jnp.min(d, axis=0, keepdims=True))

    @pl.when(r == nr - 1)
    def _finish_batch():
        lf = fwd_ref[0, 0] / jnp.float32(N)
        vsum = jnp.sum(val)
        lb = jnp.sum(colmin_ref[...] * val) / jnp.maximum(vsum, 1.0)
        out_ref[0, 0] += (jnp.float32(2.0) / B) * (lf + lb)


def kernel(pts3d_xyz, target_pts3d, target_valid):
    gt_t = jnp.swapaxes(target_pts3d, 1, 2)
    val_f = target_valid.astype(jnp.float32)[:, None, :]

    out = pl.pallas_call(
        _chamfer_body,
        grid=(B, NR),
        in_specs=[
            pl.BlockSpec((1, TILE_R, 3), lambda b, r: (b, r, 0)),
            pl.BlockSpec((1, 3, N), lambda b, r: (b, 0, 0)),
            pl.BlockSpec((1, 1, N), lambda b, r: (b, 0, 0)),
        ],
        out_specs=pl.BlockSpec(memory_space=pltpu.SMEM),
        out_shape=jax.ShapeDtypeStruct((1, 1), jnp.float32),
        scratch_shapes=[
            pltpu.VMEM((1, N), jnp.float32),
            pltpu.SMEM((1, 1), jnp.float32),
        ],
        compiler_params=pltpu.CompilerParams(
            dimension_semantics=("arbitrary", "arbitrary"),
        ),
    )(pts3d_xyz, gt_t, val_f)
    return out[0, 0]
